# Initial kernel scaffold; baseline (speedup 1.0000x reference)
import jax
import jax.numpy as jnp
from jax.experimental import pallas as pl


def kernel(pred, target):
    raise NotImplementedError("write your pallas kernel here")


# speedup vs baseline: 1.3308x; 1.3308x over previous
import jax
import jax.numpy as jnp
from jax.experimental import pallas as pl
from jax.experimental.pallas import tpu as pltpu

_C = 100000
_N = 1024
_R = 8
_SMOOTH = 0.1
_CONF = 1.0 - _SMOOTH
_EPS = _SMOOTH / (_C - 1)


def _loss_body(tgt_ref, pred_ref, out_ref):
    i = pl.program_id(0)
    x = pred_ref[...]
    m = jnp.max(x, axis=1, keepdims=True)
    se = jnp.sum(jnp.exp(x - m), axis=1, keepdims=True)
    lse = m + jnp.log(se)
    sx = jnp.sum(x, axis=1, keepdims=True)

    col = jax.lax.broadcasted_iota(jnp.int32, (1, _C), 1)
    pts = []
    for r in range(_R):
        t = tgt_ref[i * _R + r]
        xr = x[r:r + 1, :]
        pts.append(jnp.sum(jnp.where(col == t, xr, 0.0), axis=1, keepdims=True))
    pt = jnp.concatenate(pts, axis=0)

    row_loss = _EPS * (_C * lse - sx) + (_CONF - _EPS) * (lse - pt)
    part = jnp.sum(row_loss) * (1.0 / _N)

    @pl.when(i == 0)
    def _init():
        out_ref[0, 0] = 0.0

    out_ref[0, 0] += part


def kernel(pred, target):
    tgt = target.astype(jnp.int32)
    grid_spec = pltpu.PrefetchScalarGridSpec(
        num_scalar_prefetch=1,
        grid=(_N // _R,),
        in_specs=[pl.BlockSpec((_R, _C), lambda i, tref: (i, 0))],
        out_specs=pl.BlockSpec(memory_space=pltpu.SMEM,
                               block_shape=(1, 1),
                               index_map=lambda i, tref: (0, 0)),
    )
    out = pl.pallas_call(
        _loss_body,
        grid_spec=grid_spec,
        out_shape=jax.ShapeDtypeStruct((1, 1), jnp.float32),
        compiler_params=pltpu.CompilerParams(
            dimension_semantics=("arbitrary",),
        ),
    )(tgt, pred)
    return out[0, 0]
